# Initial kernel scaffold; baseline (speedup 1.0000x reference)
import jax
import jax.numpy as jnp
from jax.experimental import pallas as pl


def kernel(pc1, pc2, feat1, feat2, knn1, knn2, W11, b11, W22, b22, Wpos, bpos, gn0_gamma, gn0_beta, Wm, bm, gm_gamma, gm_beta):
    raise NotImplementedError("write your pallas kernel here")


# speedup vs baseline: 15.1886x; 15.1886x over previous
import functools

import jax
import jax.numpy as jnp
from jax import lax
from jax.experimental import pallas as pl
from jax.experimental.pallas import tpu as pltpu
from jax.experimental.pallas import tpu_sc as plsc

N = 4096
C = 64
K = 32
BD = 4
TPREP = 1024
TKNN = 256
TM = 256
TD = 128
CNT = 16 * N * K
F32 = jnp.float32
HI = lax.Precision.HIGHEST


def _dotg(a, b):
    return lax.dot_general(a, b, (((1,), (0,)), ((), ())),
                           precision=HI, preferred_element_type=F32)


def _dotd(a, b):
    return lax.dot_general(a, b, (((1,), (0,)), ((), ())),
                           precision=lax.Precision.DEFAULT,
                           preferred_element_type=F32)


def _dotd_t(a, b):
    return lax.dot_general(a, b, (((1,), (1,)), ((), ())),
                           precision=lax.Precision.DEFAULT,
                           preferred_element_type=F32)


def _leaky(x):
    return jnp.where(x >= 0, x, 0.1 * x)


def _eg():
    r = lax.broadcasted_iota(jnp.int32, (64, 4), 0) // 16
    g = lax.broadcasted_iota(jnp.int32, (64, 4), 1)
    return (r == g).astype(F32)


def _egt():
    g = lax.broadcasted_iota(jnp.int32, (4, 64), 0)
    r = lax.broadcasted_iota(jnp.int32, (4, 64), 1) // 16
    return (g == r).astype(F32)


def _prep_body(ftab_ref, fp1_ref, knn_ref, pc_ref, w11t_ref, w22t_ref,
               b11_ref, b22_ref, tab_ref, p1s_ref, knn_n_ref):
    ftab = ftab_ref[0]
    p2 = _dotd(ftab, w22t_ref[...]) + b22_ref[...]
    pc = pc_ref[0]
    pad = jnp.zeros((p2.shape[0], TD - 67), F32)
    tab_ref[0] = jnp.concatenate([p2, pc, pad], axis=1)
    p1s_ref[0] = _dotd(fp1_ref[0], w11t_ref[...]) + b11_ref[...]
    k = knn_ref[0]
    nrm = jnp.sqrt(jnp.sum(k * k, axis=1, keepdims=True)) + 1e-8
    knn_n_ref[0] = k / nrm


def _prep_call(featsT, knnsT, pcsT, w11t, w22t, b11, b22):
    grid = (BD, N // TPREP)
    return pl.pallas_call(
        _prep_body,
        grid=grid,
        in_specs=[
            pl.BlockSpec((1, TPREP, C), lambda bd, t: ((bd + 2) % 4, t, 0)),
            pl.BlockSpec((1, TPREP, C), lambda bd, t: (bd, t, 0)),
            pl.BlockSpec((1, TPREP, C), lambda bd, t: (bd, t, 0)),
            pl.BlockSpec((1, TPREP, 3), lambda bd, t: ((bd + 2) % 4, t, 0)),
            pl.BlockSpec((C, C), lambda bd, t: (0, 0)),
            pl.BlockSpec((C, C), lambda bd, t: (0, 0)),
            pl.BlockSpec((1, C), lambda bd, t: (0, 0)),
            pl.BlockSpec((1, C), lambda bd, t: (0, 0)),
        ],
        out_specs=[
            pl.BlockSpec((1, TPREP, TD), lambda bd, t: (bd, t, 0)),
            pl.BlockSpec((1, TPREP, C), lambda bd, t: (bd, t, 0)),
            pl.BlockSpec((1, TPREP, C), lambda bd, t: (bd, t, 0)),
        ],
        out_shape=[
            jax.ShapeDtypeStruct((BD, N, TD), F32),
            jax.ShapeDtypeStruct((BD, N, C), F32),
            jax.ShapeDtypeStruct((BD, N, C), F32),
        ],
    )(featsT, featsT, knnsT, pcsT, w11t, w22t, b11, b22)


def _top16_min(a):
    t_, w = a.shape
    iota = lax.broadcasted_iota(jnp.int32, (t_, w), 1)
    cols = []
    for _ in range(16):
        i = jnp.argmin(a, axis=1)[:, None]
        cols.append(i)
        a = jnp.where(iota == i, jnp.inf, a)
    return jnp.concatenate(cols, axis=1)


def _knn_body(a_ref, b_ref, ax_ref, bx_ref, idx_ref):
    bd = pl.program_id(0)
    sim = _dotd_t(a_ref[0], b_ref[0])
    ax = ax_ref[0]
    bx = bx_ref[0]
    cross = _dotd_t(ax, bx)
    sa = jnp.sum(ax * ax, axis=1, keepdims=True)
    sb = jnp.sum(bx * bx, axis=1)[None, :]
    d2 = sa + sb - 2.0 * cross
    pd = jnp.sqrt(jnp.maximum(d2, 1e-12))
    pidx = _top16_min(pd)
    fidx = _top16_min(1.0 - sim)
    idx_ref[0] = jnp.concatenate([pidx, fidx], axis=1) + bd * N


def _knn_call(knn_n, pcsT):
    grid = (BD, N // TKNN)
    return pl.pallas_call(
        _knn_body,
        grid=grid,
        in_specs=[
            pl.BlockSpec((1, TKNN, C), lambda bd, t: (bd, t, 0)),
            pl.BlockSpec((1, N, C), lambda bd, t: ((bd + 2) % 4, 0, 0)),
            pl.BlockSpec((1, TKNN, 3), lambda bd, t: (bd, t, 0)),
            pl.BlockSpec((1, N, 3), lambda bd, t: ((bd + 2) % 4, 0, 0)),
        ],
        out_specs=pl.BlockSpec((1, TKNN, K), lambda bd, t: (bd, t, 0)),
        out_shape=jax.ShapeDtypeStruct((BD, N, K), jnp.int32),
    )(knn_n, knn_n, pcsT, pcsT)


def _sc_gather(tab, idx):
    numi = idx.shape[0]
    nw = 32
    per_w = numi // nw
    win = 128
    mesh = plsc.VectorSubcoreMesh(core_axis_name="c", subcore_axis_name="s")

    @functools.partial(
        pl.kernel, mesh=mesh,
        out_type=jax.ShapeDtypeStruct((numi, TD), F32),
        scratch_types=[
            pltpu.VMEM((win,), jnp.int32),
            pltpu.VMEM((win, TD), F32),
            pltpu.SemaphoreType.DMA,
        ],
    )
    def gk(tab_hbm, idx_hbm, out_hbm, idx_v, rows_v, sem):
        wid = lax.axis_index("s") * 2 + lax.axis_index("c")
        base = wid * per_w

        @pl.loop(0, per_w // win)
        def _(w):
            off = base + w * win
            pltpu.sync_copy(idx_hbm.at[pl.ds(off, win)], idx_v)
            pltpu.async_copy(tab_hbm.at[idx_v], rows_v, sem).wait()
            pltpu.sync_copy(rows_v, out_hbm.at[pl.ds(off, win)])

    return gk(tab, idx)


def _x1_compute(g80, p1, ax, wpos_t, bpos):
    g = g80[:, 0:64]
    gx = g80[:, 64:67]
    tm = p1.shape[0]
    tk = tm * K
    axr = jnp.reshape(jnp.broadcast_to(ax[:, None, :], (tm, K, 3)), (tk, 3))
    d = _dotd(gx - axr, wpos_t) + bpos
    p1r = jnp.reshape(jnp.broadcast_to(p1[:, None, :], (tm, K, C)), (tk, C))
    return g + p1r + d


def _gstats(x):
    s = jnp.sum(x, axis=0)[None, :]
    sq = jnp.sum(x * x, axis=0)[None, :]
    eg = _eg()
    sg = _dotg(s, eg)
    qg = _dotg(sq, eg)
    return jnp.concatenate([sg, qg], axis=1)


def _stats_update(stats_ref, vec8, t):
    v = jnp.concatenate([vec8, jnp.zeros((1, 120), F32)], axis=1)
    upd = jnp.broadcast_to(v, (8, 128))

    @pl.when(t == 0)
    def _():
        stats_ref[0] = jnp.zeros((8, 128), F32)

    stats_ref[0] += upd


def _gn_mult_add(stats_row, gamma, beta):
    sg = stats_row[:, 0:4]
    qg = stats_row[:, 4:8]
    m = sg / CNT
    var = qg / CNT - m * m
    a = lax.rsqrt(var + 1e-5)
    egt = _egt()
    m_c = _dotg(m, egt)
    a_c = _dotg(a, egt)
    mult = a_c * gamma
    add = beta - m_c * mult
    return mult, add


def _stats0_body(g_ref, p1_ref, ax_ref, wposT_ref, bpos_ref, s0_ref):
    t = pl.program_id(1)
    x1 = _x1_compute(g_ref[0], p1_ref[0], ax_ref[0], wposT_ref[...],
                     bpos_ref[...])
    _stats_update(s0_ref, _gstats(x1), t)


def _stats1_body(g_ref, p1_ref, ax_ref, wposT_ref, bpos_ref, s0_ref,
                 g0g_ref, g0b_ref, wmT_ref, bm_ref, s1_ref):
    t = pl.program_id(1)
    x1 = _x1_compute(g_ref[0], p1_ref[0], ax_ref[0], wposT_ref[...],
                     bpos_ref[...])
    mult0, add0 = _gn_mult_add(s0_ref[0, 0:1, :], g0g_ref[...], g0b_ref[...])
    y1 = _leaky(x1 * mult0 + add0)
    x2 = _dotd(y1, wmT_ref[...]) + bm_ref[...]
    _stats_update(s1_ref, _gstats(x2), t)


def _final_body(g_ref, p1_ref, ax_ref, wposT_ref, bpos_ref, s0_ref,
                g0g_ref, g0b_ref, wmT_ref, bm_ref, s1_ref, gmg_ref,
                gmb_ref, out_ref):
    x1 = _x1_compute(g_ref[0], p1_ref[0], ax_ref[0], wposT_ref[...],
                     bpos_ref[...])
    mult0, add0 = _gn_mult_add(s0_ref[0, 0:1, :], g0g_ref[...], g0b_ref[...])
    y1 = _leaky(x1 * mult0 + add0)
    x2 = _dotd(y1, wmT_ref[...]) + bm_ref[...]
    mult1, add1 = _gn_mult_add(s1_ref[0, 0:1, :], gmg_ref[...], gmb_ref[...])
    y2 = _leaky(x2 * mult1 + add1)
    y3 = jnp.reshape(y2, (p1_ref.shape[1], K, C))
    out_ref[0] = jnp.max(y3, axis=1)


def _mlp_specs():
    small = lambda shape: pl.BlockSpec(shape, lambda bd, t: tuple(0 for _ in shape))
    return [
        pl.BlockSpec((1, TM * K, TD), lambda bd, t: (bd, t, 0)),
        pl.BlockSpec((1, TM, C), lambda bd, t: (bd, t, 0)),
        pl.BlockSpec((1, TM, 3), lambda bd, t: (bd, t, 0)),
        small((3, C)),
        small((1, C)),
    ]


def _stats_spec():
    return pl.BlockSpec((1, 8, 128), lambda bd, t: (bd, 0, 0))


def _mlp_calls(grows, p1s, pcsT_a, wposT, bpos, g0g, g0b, wmT, bm, gmg, gmb):
    grid = (BD, N // TM)
    stats_shape = jax.ShapeDtypeStruct((BD, 8, 128), F32)
    small = lambda shape: pl.BlockSpec(shape, lambda bd, t: tuple(0 for _ in shape))

    s0 = pl.pallas_call(
        _stats0_body, grid=grid,
        in_specs=_mlp_specs(),
        out_specs=_stats_spec(),
        out_shape=stats_shape,
    )(grows, p1s, pcsT_a, wposT, bpos)

    s1 = pl.pallas_call(
        _stats1_body, grid=grid,
        in_specs=_mlp_specs() + [_stats_spec(), small((1, C)), small((1, C)),
                                 small((C, C)), small((1, C))],
        out_specs=_stats_spec(),
        out_shape=stats_shape,
    )(grows, p1s, pcsT_a, wposT, bpos, s0, g0g, g0b, wmT, bm)

    out = pl.pallas_call(
        _final_body, grid=grid,
        in_specs=_mlp_specs() + [_stats_spec(), small((1, C)), small((1, C)),
                                 small((C, C)), small((1, C)),
                                 _stats_spec(), small((1, C)), small((1, C))],
        out_specs=pl.BlockSpec((1, TM, C), lambda bd, t: (bd, t, 0)),
        out_shape=jax.ShapeDtypeStruct((BD, N, C), F32),
    )(grows, p1s, pcsT_a, wposT, bpos, s0, g0g, g0b, wmT, bm, s1, gmg, gmb)
    return out


def kernel(pc1, pc2, feat1, feat2, knn1, knn2, W11, b11, W22, b22, Wpos,
           bpos, gn0_gamma, gn0_beta, Wm, bm, gm_gamma, gm_beta):
    featsT = jnp.concatenate([feat1, feat2], axis=0).transpose(0, 2, 1)
    knnsT = jnp.concatenate([knn1, knn2], axis=0).transpose(0, 2, 1)
    pcsT = jnp.concatenate([pc1, pc2], axis=0).transpose(0, 2, 1)

    table, p1s, knn_n = _prep_call(
        featsT, knnsT, pcsT, W11.T, W22.T,
        b11.reshape(1, C), b22.reshape(1, C))

    idx = _knn_call(knn_n, pcsT)

    g = _sc_gather(table.reshape(BD * N, TD), idx.reshape(BD * N * K))
    grows = g.reshape(BD, N * K, TD)

    out = _mlp_calls(
        grows, p1s, pcsT, Wpos.T, bpos.reshape(1, C),
        gn0_gamma.reshape(1, C), gn0_beta.reshape(1, C), Wm.T,
        bm.reshape(1, C), gm_gamma.reshape(1, C), gm_beta.reshape(1, C))

    o = out.transpose(0, 2, 1)
    return o[0:2], o[2:4]


# speedup vs baseline: 18.5856x; 1.2236x over previous
import functools

import jax
import jax.numpy as jnp
from jax import lax
from jax.experimental import pallas as pl
from jax.experimental.pallas import tpu as pltpu
from jax.experimental.pallas import tpu_sc as plsc

N = 4096
C = 64
K = 32
BD = 4
TPREP = 1024
TKNN = 256
TM = 256
TD = 128
CNT = 16 * N * K
F32 = jnp.float32
HI = lax.Precision.HIGHEST


def _dotg(a, b):
    return lax.dot_general(a, b, (((1,), (0,)), ((), ())),
                           precision=HI, preferred_element_type=F32)


def _dotd(a, b):
    return lax.dot_general(a, b, (((1,), (0,)), ((), ())),
                           precision=lax.Precision.DEFAULT,
                           preferred_element_type=F32)


def _dotd_t(a, b):
    return lax.dot_general(a, b, (((1,), (1,)), ((), ())),
                           precision=lax.Precision.DEFAULT,
                           preferred_element_type=F32)


def _leaky(x):
    return jnp.where(x >= 0, x, 0.1 * x)


def _eg():
    r = lax.broadcasted_iota(jnp.int32, (64, 4), 0) // 16
    g = lax.broadcasted_iota(jnp.int32, (64, 4), 1)
    return (r == g).astype(F32)


def _egt():
    g = lax.broadcasted_iota(jnp.int32, (4, 64), 0)
    r = lax.broadcasted_iota(jnp.int32, (4, 64), 1) // 16
    return (g == r).astype(F32)


def _prep_body(ftab_ref, fp1_ref, knn_ref, pc_ref, w11t_ref, w22t_ref,
               b11_ref, b22_ref, tab_ref, p1s_ref, knn_n_ref):
    ftab = ftab_ref[0]
    p2 = _dotd(ftab, w22t_ref[...]) + b22_ref[...]
    pc = pc_ref[0]
    pad = jnp.zeros((p2.shape[0], TD - 67), F32)
    tab_ref[0] = jnp.concatenate([p2, pc, pad], axis=1)
    p1s_ref[0] = _dotd(fp1_ref[0], w11t_ref[...]) + b11_ref[...]
    k = knn_ref[0]
    nrm = jnp.sqrt(jnp.sum(k * k, axis=1, keepdims=True)) + 1e-8
    knn_n_ref[0] = k / nrm


def _prep_call(featsT, knnsT, pcsT, w11t, w22t, b11, b22):
    grid = (BD, N // TPREP)
    return pl.pallas_call(
        _prep_body,
        grid=grid,
        in_specs=[
            pl.BlockSpec((1, TPREP, C), lambda bd, t: ((bd + 2) % 4, t, 0)),
            pl.BlockSpec((1, TPREP, C), lambda bd, t: (bd, t, 0)),
            pl.BlockSpec((1, TPREP, C), lambda bd, t: (bd, t, 0)),
            pl.BlockSpec((1, TPREP, 3), lambda bd, t: ((bd + 2) % 4, t, 0)),
            pl.BlockSpec((C, C), lambda bd, t: (0, 0)),
            pl.BlockSpec((C, C), lambda bd, t: (0, 0)),
            pl.BlockSpec((1, C), lambda bd, t: (0, 0)),
            pl.BlockSpec((1, C), lambda bd, t: (0, 0)),
        ],
        out_specs=[
            pl.BlockSpec((1, TPREP, TD), lambda bd, t: (bd, t, 0)),
            pl.BlockSpec((1, TPREP, C), lambda bd, t: (bd, t, 0)),
            pl.BlockSpec((1, TPREP, C), lambda bd, t: (bd, t, 0)),
        ],
        out_shape=[
            jax.ShapeDtypeStruct((BD, N, TD), F32),
            jax.ShapeDtypeStruct((BD, N, C), F32),
            jax.ShapeDtypeStruct((BD, N, C), F32),
        ],
    )(featsT, featsT, knnsT, pcsT, w11t, w22t, b11, b22)


CH = 32


def _top16_min_T(x):
    w, t = x.shape
    nc = w // CH
    x3 = jnp.reshape(x, (nc, CH, t))
    iota_c = lax.broadcasted_iota(jnp.int32, (nc, CH, t), 1)
    base = lax.broadcasted_iota(jnp.int32, (nc, 1, t), 0) * CH
    vals, gidx = [], []
    for _ in range(4):
        m = jnp.min(x3, axis=1, keepdims=True)
        tt = jnp.where(x3 == m, iota_c, CH)
        i = jnp.min(tt, axis=1, keepdims=True)
        vals.append(m)
        gidx.append(base + i)
        x3 = jnp.where(tt == i, jnp.inf, x3)
    cv = jnp.reshape(jnp.concatenate(vals, axis=1), (nc * 4, t))
    ci = jnp.reshape(jnp.concatenate(gidx, axis=1), (nc * 4, t))
    iota_p = lax.broadcasted_iota(jnp.int32, (nc * 4, t), 0)
    outs = []
    for _ in range(16):
        m = jnp.min(cv, axis=0, keepdims=True)
        tt = jnp.where(cv == m, iota_p, nc * 4)
        p = jnp.min(tt, axis=0, keepdims=True)
        sel = tt == p
        g = jnp.sum(jnp.where(sel, ci, 0), axis=0, keepdims=True)
        outs.append(g)
        cv = jnp.where(sel, jnp.inf, cv)
    return jnp.concatenate(outs, axis=0)


def _knn_body(at_ref, b_ref, axt_ref, bx_ref, idx_ref):
    bd = pl.program_id(0)
    simT = _dotd(b_ref[0], at_ref[0])
    axt = axt_ref[0]
    bx = bx_ref[0]
    crossT = _dotd(bx, axt)
    sa = jnp.sum(axt * axt, axis=0, keepdims=True)
    sb = jnp.sum(bx * bx, axis=1, keepdims=True)
    d2 = sa + sb - 2.0 * crossT
    pd = jnp.sqrt(jnp.maximum(d2, 1e-12))
    pidx = _top16_min_T(pd)
    fidx = _top16_min_T(1.0 - simT)
    idx_ref[0] = jnp.concatenate([pidx, fidx], axis=0) + bd * N


def _knn_call(knn_nT, knn_n, pcs, pcsT):
    grid = (BD, N // TKNN)
    return pl.pallas_call(
        _knn_body,
        grid=grid,
        in_specs=[
            pl.BlockSpec((1, C, TKNN), lambda bd, t: (bd, 0, t)),
            pl.BlockSpec((1, N, C), lambda bd, t: ((bd + 2) % 4, 0, 0)),
            pl.BlockSpec((1, 3, TKNN), lambda bd, t: (bd, 0, t)),
            pl.BlockSpec((1, N, 3), lambda bd, t: ((bd + 2) % 4, 0, 0)),
        ],
        out_specs=pl.BlockSpec((1, K, TKNN), lambda bd, t: (bd, 0, t)),
        out_shape=jax.ShapeDtypeStruct((BD, K, N), jnp.int32),
    )(knn_nT, knn_n, pcs, pcsT)


def _sc_gather(tab, idx):
    numi = idx.shape[0]
    nw = 32
    per_w = numi // nw
    win = 128
    mesh = plsc.VectorSubcoreMesh(core_axis_name="c", subcore_axis_name="s")

    @functools.partial(
        pl.kernel, mesh=mesh,
        out_type=jax.ShapeDtypeStruct((numi, TD), F32),
        scratch_types=[
            pltpu.VMEM((win,), jnp.int32),
            pltpu.VMEM((win, TD), F32),
            pltpu.SemaphoreType.DMA,
        ],
    )
    def gk(tab_hbm, idx_hbm, out_hbm, idx_v, rows_v, sem):
        wid = lax.axis_index("s") * 2 + lax.axis_index("c")
        base = wid * per_w

        @pl.loop(0, per_w // win)
        def _(w):
            off = base + w * win
            pltpu.sync_copy(idx_hbm.at[pl.ds(off, win)], idx_v)
            pltpu.async_copy(tab_hbm.at[idx_v], rows_v, sem).wait()
            pltpu.sync_copy(rows_v, out_hbm.at[pl.ds(off, win)])

    return gk(tab, idx)


def _x1_compute(g80, p1, ax, wpos_t, bpos):
    g = g80[:, 0:64]
    gx = g80[:, 64:67]
    tm = p1.shape[0]
    tk = tm * K
    axr = jnp.reshape(jnp.broadcast_to(ax[:, None, :], (tm, K, 3)), (tk, 3))
    d = _dotd(gx - axr, wpos_t) + bpos
    p1r = jnp.reshape(jnp.broadcast_to(p1[:, None, :], (tm, K, C)), (tk, C))
    return g + p1r + d


def _gstats(x):
    s = jnp.sum(x, axis=0)[None, :]
    sq = jnp.sum(x * x, axis=0)[None, :]
    eg = _eg()
    sg = _dotg(s, eg)
    qg = _dotg(sq, eg)
    return jnp.concatenate([sg, qg], axis=1)


def _stats_update(stats_ref, vec8, t):
    v = jnp.concatenate([vec8, jnp.zeros((1, 120), F32)], axis=1)
    upd = jnp.broadcast_to(v, (8, 128))

    @pl.when(t == 0)
    def _():
        stats_ref[0] = jnp.zeros((8, 128), F32)

    stats_ref[0] += upd


def _gn_mult_add(stats_row, gamma, beta):
    sg = stats_row[:, 0:4]
    qg = stats_row[:, 4:8]
    m = sg / CNT
    var = qg / CNT - m * m
    a = lax.rsqrt(var + 1e-5)
    egt = _egt()
    m_c = _dotg(m, egt)
    a_c = _dotg(a, egt)
    mult = a_c * gamma
    add = beta - m_c * mult
    return mult, add


def _stats0_body(g_ref, p1_ref, ax_ref, wposT_ref, bpos_ref, s0_ref):
    t = pl.program_id(1)
    x1 = _x1_compute(g_ref[0], p1_ref[0], ax_ref[0], wposT_ref[...],
                     bpos_ref[...])
    _stats_update(s0_ref, _gstats(x1), t)


def _stats1_body(g_ref, p1_ref, ax_ref, wposT_ref, bpos_ref, s0_ref,
                 g0g_ref, g0b_ref, wmT_ref, bm_ref, s1_ref):
    t = pl.program_id(1)
    x1 = _x1_compute(g_ref[0], p1_ref[0], ax_ref[0], wposT_ref[...],
                     bpos_ref[...])
    mult0, add0 = _gn_mult_add(s0_ref[0, 0:1, :], g0g_ref[...], g0b_ref[...])
    y1 = _leaky(x1 * mult0 + add0)
    x2 = _dotd(y1, wmT_ref[...]) + bm_ref[...]
    _stats_update(s1_ref, _gstats(x2), t)


def _final_body(g_ref, p1_ref, ax_ref, wposT_ref, bpos_ref, s0_ref,
                g0g_ref, g0b_ref, wmT_ref, bm_ref, s1_ref, gmg_ref,
                gmb_ref, out_ref):
    x1 = _x1_compute(g_ref[0], p1_ref[0], ax_ref[0], wposT_ref[...],
                     bpos_ref[...])
    mult0, add0 = _gn_mult_add(s0_ref[0, 0:1, :], g0g_ref[...], g0b_ref[...])
    y1 = _leaky(x1 * mult0 + add0)
    x2 = _dotd(y1, wmT_ref[...]) + bm_ref[...]
    mult1, add1 = _gn_mult_add(s1_ref[0, 0:1, :], gmg_ref[...], gmb_ref[...])
    y2 = _leaky(x2 * mult1 + add1)
    y3 = jnp.reshape(y2, (p1_ref.shape[1], K, C))
    out_ref[0] = jnp.max(y3, axis=1)


def _mlp_specs():
    small = lambda shape: pl.BlockSpec(shape, lambda bd, t: tuple(0 for _ in shape))
    return [
        pl.BlockSpec((1, TM * K, TD), lambda bd, t: (bd, t, 0)),
        pl.BlockSpec((1, TM, C), lambda bd, t: (bd, t, 0)),
        pl.BlockSpec((1, TM, 3), lambda bd, t: (bd, t, 0)),
        small((3, C)),
        small((1, C)),
    ]


def _stats_spec():
    return pl.BlockSpec((1, 8, 128), lambda bd, t: (bd, 0, 0))


def _mlp_calls(grows, p1s, pcsT_a, wposT, bpos, g0g, g0b, wmT, bm, gmg, gmb):
    grid = (BD, N // TM)
    stats_shape = jax.ShapeDtypeStruct((BD, 8, 128), F32)
    small = lambda shape: pl.BlockSpec(shape, lambda bd, t: tuple(0 for _ in shape))

    s0 = pl.pallas_call(
        _stats0_body, grid=grid,
        in_specs=_mlp_specs(),
        out_specs=_stats_spec(),
        out_shape=stats_shape,
    )(grows, p1s, pcsT_a, wposT, bpos)

    s1 = pl.pallas_call(
        _stats1_body, grid=grid,
        in_specs=_mlp_specs() + [_stats_spec(), small((1, C)), small((1, C)),
                                 small((C, C)), small((1, C))],
        out_specs=_stats_spec(),
        out_shape=stats_shape,
    )(grows, p1s, pcsT_a, wposT, bpos, s0, g0g, g0b, wmT, bm)

    out = pl.pallas_call(
        _final_body, grid=grid,
        in_specs=_mlp_specs() + [_stats_spec(), small((1, C)), small((1, C)),
                                 small((C, C)), small((1, C)),
                                 _stats_spec(), small((1, C)), small((1, C))],
        out_specs=pl.BlockSpec((1, TM, C), lambda bd, t: (bd, t, 0)),
        out_shape=jax.ShapeDtypeStruct((BD, N, C), F32),
    )(grows, p1s, pcsT_a, wposT, bpos, s0, g0g, g0b, wmT, bm, s1, gmg, gmb)
    return out


def kernel(pc1, pc2, feat1, feat2, knn1, knn2, W11, b11, W22, b22, Wpos,
           bpos, gn0_gamma, gn0_beta, Wm, bm, gm_gamma, gm_beta):
    featsT = jnp.concatenate([feat1, feat2], axis=0).transpose(0, 2, 1)
    knnsT = jnp.concatenate([knn1, knn2], axis=0).transpose(0, 2, 1)
    pcsT = jnp.concatenate([pc1, pc2], axis=0).transpose(0, 2, 1)

    table, p1s, knn_n = _prep_call(
        featsT, knnsT, pcsT, W11.T, W22.T,
        b11.reshape(1, C), b22.reshape(1, C))

    pcs = jnp.concatenate([pc1, pc2], axis=0)
    idxT = _knn_call(knn_n.transpose(0, 2, 1), knn_n, pcs, pcsT)
    idx = idxT.transpose(0, 2, 1)

    g = _sc_gather(table.reshape(BD * N, TD), idx.reshape(BD * N * K))
    grows = g.reshape(BD, N * K, TD)

    out = _mlp_calls(
        grows, p1s, pcsT, Wpos.T, bpos.reshape(1, C),
        gn0_gamma.reshape(1, C), gn0_beta.reshape(1, C), Wm.T,
        bm.reshape(1, C), gm_gamma.reshape(1, C), gm_beta.reshape(1, C))

    o = out.transpose(0, 2, 1)
    return o[0:2], o[2:4]


# speedup vs baseline: 21.8440x; 1.1753x over previous
import functools

import jax
import jax.numpy as jnp
from jax import lax
from jax.experimental import pallas as pl
from jax.experimental.pallas import tpu as pltpu
from jax.experimental.pallas import tpu_sc as plsc

N = 4096
C = 64
K = 32
BD = 4
TPREP = 1024
TKNN = 256
TM = 256
TD = 128
CNT = 16 * N * K
F32 = jnp.float32
HI = lax.Precision.HIGHEST


def _dotg(a, b):
    return lax.dot_general(a, b, (((1,), (0,)), ((), ())),
                           precision=HI, preferred_element_type=F32)


def _dotd(a, b):
    return lax.dot_general(a, b, (((1,), (0,)), ((), ())),
                           precision=lax.Precision.DEFAULT,
                           preferred_element_type=F32)


def _dotd_t(a, b):
    return lax.dot_general(a, b, (((1,), (1,)), ((), ())),
                           precision=lax.Precision.DEFAULT,
                           preferred_element_type=F32)


def _leaky(x):
    return jnp.where(x >= 0, x, 0.1 * x)


def _eg():
    r = lax.broadcasted_iota(jnp.int32, (64, 4), 0) // 16
    g = lax.broadcasted_iota(jnp.int32, (64, 4), 1)
    return (r == g).astype(F32)


def _egt():
    g = lax.broadcasted_iota(jnp.int32, (4, 64), 0)
    r = lax.broadcasted_iota(jnp.int32, (4, 64), 1) // 16
    return (g == r).astype(F32)


def _prep_body(ftab_ref, fp1_ref, knn_ref, pc_ref, w11t_ref, w22t_ref,
               b11_ref, b22_ref, tab_ref, p1s_ref, knn_n_ref):
    ftab = ftab_ref[0]
    p2 = _dotd(ftab, w22t_ref[...]) + b22_ref[...]
    pc = pc_ref[0]
    pad = jnp.zeros((p2.shape[0], TD - 67), F32)
    tab_ref[0] = jnp.concatenate([p2, pc, pad], axis=1)
    p1s_ref[0] = _dotd(fp1_ref[0], w11t_ref[...]) + b11_ref[...]
    k = knn_ref[0]
    nrm = jnp.sqrt(jnp.sum(k * k, axis=1, keepdims=True)) + 1e-8
    knn_n_ref[0] = k / nrm


def _prep_call(featsT, knnsT, pcsT, w11t, w22t, b11, b22):
    grid = (BD, N // TPREP)
    return pl.pallas_call(
        _prep_body,
        grid=grid,
        in_specs=[
            pl.BlockSpec((1, TPREP, C), lambda bd, t: ((bd + 2) % 4, t, 0)),
            pl.BlockSpec((1, TPREP, C), lambda bd, t: (bd, t, 0)),
            pl.BlockSpec((1, TPREP, C), lambda bd, t: (bd, t, 0)),
            pl.BlockSpec((1, TPREP, 3), lambda bd, t: ((bd + 2) % 4, t, 0)),
            pl.BlockSpec((C, C), lambda bd, t: (0, 0)),
            pl.BlockSpec((C, C), lambda bd, t: (0, 0)),
            pl.BlockSpec((1, C), lambda bd, t: (0, 0)),
            pl.BlockSpec((1, C), lambda bd, t: (0, 0)),
        ],
        out_specs=[
            pl.BlockSpec((1, TPREP, TD), lambda bd, t: (bd, t, 0)),
            pl.BlockSpec((1, TPREP, C), lambda bd, t: (bd, t, 0)),
            pl.BlockSpec((1, TPREP, C), lambda bd, t: (bd, t, 0)),
        ],
        out_shape=[
            jax.ShapeDtypeStruct((BD, N, TD), F32),
            jax.ShapeDtypeStruct((BD, N, C), F32),
            jax.ShapeDtypeStruct((BD, N, C), F32),
        ],
    )(featsT, featsT, knnsT, pcsT, w11t, w22t, b11, b22)


CH = 32


def _top16_min_T(x):
    w, t = x.shape
    nc = w // CH
    x3 = jnp.reshape(x, (nc, CH, t))
    iota_c = lax.broadcasted_iota(jnp.int32, (nc, CH, t), 1)
    base = lax.broadcasted_iota(jnp.int32, (nc, 1, t), 0) * CH
    vals, gidx = [], []
    for _ in range(4):
        m = jnp.min(x3, axis=1, keepdims=True)
        tt = jnp.where(x3 == m, iota_c, CH)
        i = jnp.min(tt, axis=1, keepdims=True)
        vals.append(m)
        gidx.append(base + i)
        x3 = jnp.where(tt == i, jnp.inf, x3)
    cv = jnp.reshape(jnp.concatenate(vals, axis=1), (nc * 4, t))
    ci = jnp.reshape(jnp.concatenate(gidx, axis=1), (nc * 4, t))
    iota_p = lax.broadcasted_iota(jnp.int32, (nc * 4, t), 0)
    outs = []
    for _ in range(16):
        m = jnp.min(cv, axis=0, keepdims=True)
        tt = jnp.where(cv == m, iota_p, nc * 4)
        p = jnp.min(tt, axis=0, keepdims=True)
        sel = tt == p
        g = jnp.sum(jnp.where(sel, ci, 0), axis=0, keepdims=True)
        outs.append(g)
        cv = jnp.where(sel, jnp.inf, cv)
    return jnp.concatenate(outs, axis=0)


def _knn_body(at_ref, b_ref, axt_ref, bx_ref, idx_ref, *, bd_off):
    bd = pl.program_id(0)
    simT = _dotd(b_ref[0], at_ref[0])
    axt = axt_ref[0]
    bx = bx_ref[0]
    crossT = _dotd(bx, axt)
    sa = jnp.sum(axt * axt, axis=0, keepdims=True)
    sb = jnp.sum(bx * bx, axis=1, keepdims=True)
    d2 = sa + sb - 2.0 * crossT
    pd = jnp.sqrt(jnp.maximum(d2, 1e-12))
    pidx = _top16_min_T(pd)
    fidx = _top16_min_T(1.0 - simT)
    idx_ref[0] = jnp.concatenate([pidx, fidx], axis=0) + (bd + bd_off) * N


def _knn_call(knn_nT_a, knn_n_b, pcs_a, pcsT_b, bd_off):
    nbd = knn_nT_a.shape[0]
    grid = (nbd, N // TKNN)
    return pl.pallas_call(
        functools.partial(_knn_body, bd_off=bd_off),
        grid=grid,
        in_specs=[
            pl.BlockSpec((1, C, TKNN), lambda bd, t: (bd, 0, t)),
            pl.BlockSpec((1, N, C), lambda bd, t: (bd, 0, 0)),
            pl.BlockSpec((1, 3, TKNN), lambda bd, t: (bd, 0, t)),
            pl.BlockSpec((1, N, 3), lambda bd, t: (bd, 0, 0)),
        ],
        out_specs=pl.BlockSpec((1, K, TKNN), lambda bd, t: (bd, 0, t)),
        out_shape=jax.ShapeDtypeStruct((nbd, K, N), jnp.int32),
    )(knn_nT_a, knn_n_b, pcs_a, pcsT_b)


def _sc_gather(tab, idx):
    numi = idx.shape[0]
    nw = 32
    per_w = numi // nw
    win = 128
    mesh = plsc.VectorSubcoreMesh(core_axis_name="c", subcore_axis_name="s")

    @functools.partial(
        pl.kernel, mesh=mesh,
        out_type=jax.ShapeDtypeStruct((numi, TD), F32),
        scratch_types=[
            pltpu.VMEM((win,), jnp.int32),
            pltpu.VMEM((win, TD), F32),
            pltpu.SemaphoreType.DMA,
        ],
    )
    def gk(tab_hbm, idx_hbm, out_hbm, idx_v, rows_v, sem):
        wid = lax.axis_index("s") * 2 + lax.axis_index("c")
        base = wid * per_w

        @pl.loop(0, per_w // win)
        def _(w):
            off = base + w * win
            pltpu.sync_copy(idx_hbm.at[pl.ds(off, win)], idx_v)
            pltpu.async_copy(tab_hbm.at[idx_v], rows_v, sem).wait()
            pltpu.sync_copy(rows_v, out_hbm.at[pl.ds(off, win)])

    return gk(tab, idx)


def _x1_compute(g80, p1, ax, wpos_t, bpos):
    g = g80[:, 0:64]
    gx = g80[:, 64:67]
    tm = p1.shape[0]
    tk = tm * K
    axr = jnp.reshape(jnp.broadcast_to(ax[:, None, :], (tm, K, 3)), (tk, 3))
    d = _dotd(gx - axr, wpos_t) + bpos
    p1r = jnp.reshape(jnp.broadcast_to(p1[:, None, :], (tm, K, C)), (tk, C))
    return g + p1r + d


def _gstats(x):
    s = jnp.sum(x, axis=0)[None, :]
    sq = jnp.sum(x * x, axis=0)[None, :]
    eg = _eg()
    sg = _dotg(s, eg)
    qg = _dotg(sq, eg)
    return jnp.concatenate([sg, qg], axis=1)


def _stats_update(stats_ref, vec8, t):
    v = jnp.concatenate([vec8, jnp.zeros((1, 120), F32)], axis=1)
    upd = jnp.broadcast_to(v, (8, 128))

    @pl.when(t == 0)
    def _():
        stats_ref[0] = jnp.zeros((8, 128), F32)

    stats_ref[0] += upd


def _gn_mult_add(stats_row, gamma, beta):
    sg = stats_row[:, 0:4]
    qg = stats_row[:, 4:8]
    m = sg / CNT
    var = qg / CNT - m * m
    a = lax.rsqrt(var + 1e-5)
    egt = _egt()
    m_c = _dotg(m, egt)
    a_c = _dotg(a, egt)
    mult = a_c * gamma
    add = beta - m_c * mult
    return mult, add


def _stats0_body(g_ref, p1_ref, ax_ref, wposT_ref, bpos_ref, s0_ref):
    t = pl.program_id(1)
    x1 = _x1_compute(g_ref[0], p1_ref[0], ax_ref[0], wposT_ref[...],
                     bpos_ref[...])
    _stats_update(s0_ref, _gstats(x1), t)


def _stats1_body(g_ref, p1_ref, ax_ref, wposT_ref, bpos_ref, s0_ref,
                 g0g_ref, g0b_ref, wmT_ref, bm_ref, s1_ref, y1_ref):
    t = pl.program_id(1)
    x1 = _x1_compute(g_ref[0], p1_ref[0], ax_ref[0], wposT_ref[...],
                     bpos_ref[...])
    mult0, add0 = _gn_mult_add(s0_ref[0, 0:1, :], g0g_ref[...], g0b_ref[...])
    y1 = _leaky(x1 * mult0 + add0)
    y1_ref[0] = y1.astype(jnp.bfloat16)
    x2 = _dotd(y1, wmT_ref[...]) + bm_ref[...]
    _stats_update(s1_ref, _gstats(x2), t)


def _final_body(y1_ref, wmT_ref, bm_ref, s1_ref, gmg_ref, gmb_ref, out_ref):
    y1 = y1_ref[0].astype(F32)
    x2 = _dotd(y1, wmT_ref[...]) + bm_ref[...]
    mult1, add1 = _gn_mult_add(s1_ref[0, 0:1, :], gmg_ref[...], gmb_ref[...])
    y2 = _leaky(x2 * mult1 + add1)
    y3 = jnp.reshape(y2, (TM, K, C))
    out_ref[0] = jnp.max(y3, axis=1)


def _mlp_specs():
    small = lambda shape: pl.BlockSpec(shape, lambda bd, t: tuple(0 for _ in shape))
    return [
        pl.BlockSpec((1, TM * K, TD), lambda bd, t: (bd, t, 0)),
        pl.BlockSpec((1, TM, C), lambda bd, t: (bd, t, 0)),
        pl.BlockSpec((1, TM, 3), lambda bd, t: (bd, t, 0)),
        small((3, C)),
        small((1, C)),
    ]


def _stats_spec():
    return pl.BlockSpec((1, 8, 128), lambda bd, t: (bd, 0, 0))


def _mlp_calls(grows, p1s, pcsT_a, wposT, bpos, g0g, g0b, wmT, bm, gmg, gmb):
    nbd = grows.shape[0]
    grid = (nbd, N // TM)
    stats_shape = jax.ShapeDtypeStruct((nbd, 8, 128), F32)
    small = lambda shape: pl.BlockSpec(shape, lambda bd, t: tuple(0 for _ in shape))

    s0 = pl.pallas_call(
        _stats0_body, grid=grid,
        in_specs=_mlp_specs(),
        out_specs=_stats_spec(),
        out_shape=stats_shape,
    )(grows, p1s, pcsT_a, wposT, bpos)

    y1_spec = pl.BlockSpec((1, TM * K, C), lambda bd, t: (bd, t, 0))
    s1, y1st = pl.pallas_call(
        _stats1_body, grid=grid,
        in_specs=_mlp_specs() + [_stats_spec(), small((1, C)), small((1, C)),
                                 small((C, C)), small((1, C))],
        out_specs=[_stats_spec(), y1_spec],
        out_shape=[stats_shape,
                   jax.ShapeDtypeStruct((nbd, N * K, C), jnp.bfloat16)],
    )(grows, p1s, pcsT_a, wposT, bpos, s0, g0g, g0b, wmT, bm)

    out = pl.pallas_call(
        _final_body, grid=grid,
        in_specs=[y1_spec, small((C, C)), small((1, C)),
                  _stats_spec(), small((1, C)), small((1, C))],
        out_specs=pl.BlockSpec((1, TM, C), lambda bd, t: (bd, t, 0)),
        out_shape=jax.ShapeDtypeStruct((nbd, N, C), F32),
    )(y1st, wmT, bm, s1, gmg, gmb)
    return out


def kernel(pc1, pc2, feat1, feat2, knn1, knn2, W11, b11, W22, b22, Wpos,
           bpos, gn0_gamma, gn0_beta, Wm, bm, gm_gamma, gm_beta):
    featsT = jnp.concatenate([feat1, feat2], axis=0).transpose(0, 2, 1)
    knnsT = jnp.concatenate([knn1, knn2], axis=0).transpose(0, 2, 1)
    pcsT = jnp.concatenate([pc1, pc2], axis=0).transpose(0, 2, 1)

    table, p1s, knn_n = _prep_call(
        featsT, knnsT, pcsT, W11.T, W22.T,
        b11.reshape(1, C), b22.reshape(1, C))

    pcs = jnp.concatenate([pc1, pc2], axis=0)
    knn_nT = knn_n.transpose(0, 2, 1)
    flat_tab = table.reshape(BD * N, TD)

    outs = []
    for h in (0, 1):
        a_sl = slice(2 * h, 2 * h + 2)
        b_sl = slice(2 - 2 * h, 4 - 2 * h)
        idxT = _knn_call(knn_nT[a_sl], knn_n[b_sl], pcs[a_sl], pcsT[b_sl],
                         2 * h)
        idx = idxT.transpose(0, 2, 1)
        g = _sc_gather(flat_tab, idx.reshape(2 * N * K))
        grows = g.reshape(2, N * K, TD)
        outs.append(_mlp_calls(
            grows, p1s[a_sl], pcsT[a_sl], Wpos.T, bpos.reshape(1, C),
            gn0_gamma.reshape(1, C), gn0_beta.reshape(1, C), Wm.T,
            bm.reshape(1, C), gm_gamma.reshape(1, C), gm_beta.reshape(1, C)))

    o0 = outs[0].transpose(0, 2, 1)
    o1 = outs[1].transpose(0, 2, 1)
    return o0, o1


# speedup vs baseline: 21.8510x; 1.0003x over previous
import functools

import jax
import jax.numpy as jnp
from jax import lax
from jax.experimental import pallas as pl
from jax.experimental.pallas import tpu as pltpu
from jax.experimental.pallas import tpu_sc as plsc

N = 4096
C = 64
K = 32
BD = 4
TPREP = 1024
TKNN = 256
TM = 256
TD = 128
CNT = 16 * N * K
F32 = jnp.float32
HI = lax.Precision.HIGHEST


def _dotg(a, b):
    return lax.dot_general(a, b, (((1,), (0,)), ((), ())),
                           precision=HI, preferred_element_type=F32)


def _dotd(a, b):
    return lax.dot_general(a, b, (((1,), (0,)), ((), ())),
                           precision=lax.Precision.DEFAULT,
                           preferred_element_type=F32)


def _dotd_t(a, b):
    return lax.dot_general(a, b, (((1,), (1,)), ((), ())),
                           precision=lax.Precision.DEFAULT,
                           preferred_element_type=F32)


def _leaky(x):
    return jnp.where(x >= 0, x, 0.1 * x)


def _eg():
    r = lax.broadcasted_iota(jnp.int32, (64, 4), 0) // 16
    g = lax.broadcasted_iota(jnp.int32, (64, 4), 1)
    return (r == g).astype(F32)


def _egt():
    g = lax.broadcasted_iota(jnp.int32, (4, 64), 0)
    r = lax.broadcasted_iota(jnp.int32, (4, 64), 1) // 16
    return (g == r).astype(F32)


def _prep_body(ftab_ref, fp1_ref, knn_ref, pc_ref, w11t_ref, w22t_ref,
               b11_ref, b22_ref, tab_ref, p1s_ref, knn_n_ref):
    ftab = ftab_ref[0]
    p2 = _dotd(ftab, w22t_ref[...]) + b22_ref[...]
    pc = pc_ref[0]
    pad = jnp.zeros((p2.shape[0], TD - 67), F32)
    tab_ref[0] = jnp.concatenate([p2, pc, pad], axis=1)
    p1s_ref[0] = _dotd(fp1_ref[0], w11t_ref[...]) + b11_ref[...]
    k = knn_ref[0]
    nrm = jnp.sqrt(jnp.sum(k * k, axis=1, keepdims=True)) + 1e-8
    knn_n_ref[0] = k / nrm


def _prep_call(featsT, knnsT, pcsT, w11t, w22t, b11, b22):
    grid = (BD, N // TPREP)
    return pl.pallas_call(
        _prep_body,
        grid=grid,
        in_specs=[
            pl.BlockSpec((1, TPREP, C), lambda bd, t: ((bd + 2) % 4, t, 0)),
            pl.BlockSpec((1, TPREP, C), lambda bd, t: (bd, t, 0)),
            pl.BlockSpec((1, TPREP, C), lambda bd, t: (bd, t, 0)),
            pl.BlockSpec((1, TPREP, 3), lambda bd, t: ((bd + 2) % 4, t, 0)),
            pl.BlockSpec((C, C), lambda bd, t: (0, 0)),
            pl.BlockSpec((C, C), lambda bd, t: (0, 0)),
            pl.BlockSpec((1, C), lambda bd, t: (0, 0)),
            pl.BlockSpec((1, C), lambda bd, t: (0, 0)),
        ],
        out_specs=[
            pl.BlockSpec((1, TPREP, TD), lambda bd, t: (bd, t, 0)),
            pl.BlockSpec((1, TPREP, C), lambda bd, t: (bd, t, 0)),
            pl.BlockSpec((1, TPREP, C), lambda bd, t: (bd, t, 0)),
        ],
        out_shape=[
            jax.ShapeDtypeStruct((BD, N, TD), F32),
            jax.ShapeDtypeStruct((BD, N, C), F32),
            jax.ShapeDtypeStruct((BD, N, C), F32),
        ],
    )(featsT, featsT, knnsT, pcsT, w11t, w22t, b11, b22)


CH = 32


def _top16_min_T(x):
    w, t = x.shape
    nc = w // CH
    x3 = jnp.reshape(x, (nc, CH, t))
    iota_c = lax.broadcasted_iota(jnp.int32, (nc, CH, t), 1)
    base = lax.broadcasted_iota(jnp.int32, (nc, 1, t), 0) * CH
    vals, gidx = [], []
    for j in range(4):
        m = jnp.min(x3, axis=1, keepdims=True)
        tt = jnp.where(x3 == m, iota_c, CH)
        i = jnp.min(tt, axis=1, keepdims=True)
        vals.append(m)
        gidx.append(base + i)
        if j < 3:
            x3 = jnp.where(tt == i, jnp.inf, x3)
    cv = jnp.reshape(jnp.concatenate(vals, axis=1), (nc * 4, t))
    ci = jnp.reshape(jnp.concatenate(gidx, axis=1), (nc * 4, t))
    iota_p = lax.broadcasted_iota(jnp.int32, (nc * 4, t), 0)
    outs = []
    for j in range(16):
        m = jnp.min(cv, axis=0, keepdims=True)
        tt = jnp.where(cv == m, iota_p, nc * 4)
        p = jnp.min(tt, axis=0, keepdims=True)
        sel = tt == p
        g = jnp.sum(jnp.where(sel, ci, 0), axis=0, keepdims=True)
        outs.append(g)
        if j < 15:
            cv = jnp.where(sel, jnp.inf, cv)
    return jnp.concatenate(outs, axis=0)


def _knn_body(at_ref, b_ref, axt_ref, bx_ref, idx_ref, *, bd_off):
    bd = pl.program_id(0)
    simT = _dotd(b_ref[0], at_ref[0])
    axt = axt_ref[0]
    bx = bx_ref[0]
    crossT = _dotd(bx, axt)
    sa = jnp.sum(axt * axt, axis=0, keepdims=True)
    sb = jnp.sum(bx * bx, axis=1, keepdims=True)
    d2 = sa + sb - 2.0 * crossT
    pd = jnp.sqrt(jnp.maximum(d2, 1e-12))
    pidx = _top16_min_T(pd)
    fidx = _top16_min_T(1.0 - simT)
    idx_ref[0] = jnp.concatenate([pidx, fidx], axis=0) + (bd + bd_off) * N


def _knn_call(knn_nT_a, knn_n_b, pcs_a, pcsT_b, bd_off):
    nbd = knn_nT_a.shape[0]
    grid = (nbd, N // TKNN)
    return pl.pallas_call(
        functools.partial(_knn_body, bd_off=bd_off),
        grid=grid,
        in_specs=[
            pl.BlockSpec((1, C, TKNN), lambda bd, t: (bd, 0, t)),
            pl.BlockSpec((1, N, C), lambda bd, t: (bd, 0, 0)),
            pl.BlockSpec((1, 3, TKNN), lambda bd, t: (bd, 0, t)),
            pl.BlockSpec((1, N, 3), lambda bd, t: (bd, 0, 0)),
        ],
        out_specs=pl.BlockSpec((1, K, TKNN), lambda bd, t: (bd, 0, t)),
        out_shape=jax.ShapeDtypeStruct((nbd, K, N), jnp.int32),
    )(knn_nT_a, knn_n_b, pcs_a, pcsT_b)


def _sc_gather(tab, idx):
    numi = idx.shape[0]
    nw = 32
    per_w = numi // nw
    win = 128
    mesh = plsc.VectorSubcoreMesh(core_axis_name="c", subcore_axis_name="s")

    @functools.partial(
        pl.kernel, mesh=mesh,
        out_type=jax.ShapeDtypeStruct((numi, TD), F32),
        scratch_types=[
            pltpu.VMEM((win,), jnp.int32),
            pltpu.VMEM((win, TD), F32),
            pltpu.SemaphoreType.DMA,
        ],
    )
    def gk(tab_hbm, idx_hbm, out_hbm, idx_v, rows_v, sem):
        wid = lax.axis_index("s") * 2 + lax.axis_index("c")
        base = wid * per_w

        @pl.loop(0, per_w // win)
        def _(w):
            off = base + w * win
            pltpu.sync_copy(idx_hbm.at[pl.ds(off, win)], idx_v)
            pltpu.async_copy(tab_hbm.at[idx_v], rows_v, sem).wait()
            pltpu.sync_copy(rows_v, out_hbm.at[pl.ds(off, win)])

    return gk(tab, idx)


def _x1_compute(g80, p1, ax, wpos_t, bpos):
    g = g80[:, 0:64]
    gx = g80[:, 64:67]
    tm = p1.shape[0]
    tk = tm * K
    axr = jnp.reshape(jnp.broadcast_to(ax[:, None, :], (tm, K, 3)), (tk, 3))
    d = _dotd(gx - axr, wpos_t) + bpos
    p1r = jnp.reshape(jnp.broadcast_to(p1[:, None, :], (tm, K, C)), (tk, C))
    return g + p1r + d


def _gstats(x):
    s = jnp.sum(x, axis=0)[None, :]
    sq = jnp.sum(x * x, axis=0)[None, :]
    eg = _eg()
    sg = _dotg(s, eg)
    qg = _dotg(sq, eg)
    return jnp.concatenate([sg, qg], axis=1)


def _stats_update(stats_ref, vec8, t):
    v = jnp.concatenate([vec8, jnp.zeros((1, 120), F32)], axis=1)
    upd = jnp.broadcast_to(v, (8, 128))

    @pl.when(t == 0)
    def _():
        stats_ref[0] = jnp.zeros((8, 128), F32)

    stats_ref[0] += upd


def _gn_mult_add(stats_row, gamma, beta):
    sg = stats_row[:, 0:4]
    qg = stats_row[:, 4:8]
    m = sg / CNT
    var = qg / CNT - m * m
    a = lax.rsqrt(var + 1e-5)
    egt = _egt()
    m_c = _dotg(m, egt)
    a_c = _dotg(a, egt)
    mult = a_c * gamma
    add = beta - m_c * mult
    return mult, add


def _stats0_body(g_ref, p1_ref, ax_ref, wposT_ref, bpos_ref, s0_ref):
    t = pl.program_id(1)
    x1 = _x1_compute(g_ref[0], p1_ref[0], ax_ref[0], wposT_ref[...],
                     bpos_ref[...])
    _stats_update(s0_ref, _gstats(x1), t)


def _stats1_body(g_ref, p1_ref, ax_ref, wposT_ref, bpos_ref, s0_ref,
                 g0g_ref, g0b_ref, wmT_ref, bm_ref, s1_ref, y1_ref):
    t = pl.program_id(1)
    x1 = _x1_compute(g_ref[0], p1_ref[0], ax_ref[0], wposT_ref[...],
                     bpos_ref[...])
    mult0, add0 = _gn_mult_add(s0_ref[0, 0:1, :], g0g_ref[...], g0b_ref[...])
    y1 = _leaky(x1 * mult0 + add0)
    y1_ref[0] = y1.astype(jnp.bfloat16)
    x2 = _dotd(y1, wmT_ref[...]) + bm_ref[...]
    _stats_update(s1_ref, _gstats(x2), t)


def _final_body(y1_ref, wmT_ref, bm_ref, s1_ref, gmg_ref, gmb_ref, out_ref):
    y1 = y1_ref[0].astype(F32)
    x2 = _dotd(y1, wmT_ref[...]) + bm_ref[...]
    mult1, add1 = _gn_mult_add(s1_ref[0, 0:1, :], gmg_ref[...], gmb_ref[...])
    y2 = _leaky(x2 * mult1 + add1)
    y3 = jnp.reshape(y2, (TM, K, C))
    out_ref[0] = jnp.max(y3, axis=1)


def _mlp_specs():
    small = lambda shape: pl.BlockSpec(shape, lambda bd, t: tuple(0 for _ in shape))
    return [
        pl.BlockSpec((1, TM * K, TD), lambda bd, t: (bd, t, 0)),
        pl.BlockSpec((1, TM, C), lambda bd, t: (bd, t, 0)),
        pl.BlockSpec((1, TM, 3), lambda bd, t: (bd, t, 0)),
        small((3, C)),
        small((1, C)),
    ]


def _stats_spec():
    return pl.BlockSpec((1, 8, 128), lambda bd, t: (bd, 0, 0))


def _mlp_calls(grows, p1s, pcsT_a, wposT, bpos, g0g, g0b, wmT, bm, gmg, gmb):
    nbd = grows.shape[0]
    grid = (nbd, N // TM)
    stats_shape = jax.ShapeDtypeStruct((nbd, 8, 128), F32)
    small = lambda shape: pl.BlockSpec(shape, lambda bd, t: tuple(0 for _ in shape))

    s0 = pl.pallas_call(
        _stats0_body, grid=grid,
        in_specs=_mlp_specs(),
        out_specs=_stats_spec(),
        out_shape=stats_shape,
    )(grows, p1s, pcsT_a, wposT, bpos)

    y1_spec = pl.BlockSpec((1, TM * K, C), lambda bd, t: (bd, t, 0))
    s1, y1st = pl.pallas_call(
        _stats1_body, grid=grid,
        in_specs=_mlp_specs() + [_stats_spec(), small((1, C)), small((1, C)),
                                 small((C, C)), small((1, C))],
        out_specs=[_stats_spec(), y1_spec],
        out_shape=[stats_shape,
                   jax.ShapeDtypeStruct((nbd, N * K, C), jnp.bfloat16)],
    )(grows, p1s, pcsT_a, wposT, bpos, s0, g0g, g0b, wmT, bm)

    out = pl.pallas_call(
        _final_body, grid=grid,
        in_specs=[y1_spec, small((C, C)), small((1, C)),
                  _stats_spec(), small((1, C)), small((1, C))],
        out_specs=pl.BlockSpec((1, TM, C), lambda bd, t: (bd, t, 0)),
        out_shape=jax.ShapeDtypeStruct((nbd, N, C), F32),
    )(y1st, wmT, bm, s1, gmg, gmb)
    return out


def kernel(pc1, pc2, feat1, feat2, knn1, knn2, W11, b11, W22, b22, Wpos,
           bpos, gn0_gamma, gn0_beta, Wm, bm, gm_gamma, gm_beta):
    featsT = jnp.concatenate([feat1, feat2], axis=0).transpose(0, 2, 1)
    knnsT = jnp.concatenate([knn1, knn2], axis=0).transpose(0, 2, 1)
    pcsT = jnp.concatenate([pc1, pc2], axis=0).transpose(0, 2, 1)

    table, p1s, knn_n = _prep_call(
        featsT, knnsT, pcsT, W11.T, W22.T,
        b11.reshape(1, C), b22.reshape(1, C))

    pcs = jnp.concatenate([pc1, pc2], axis=0)
    knn_nT = knn_n.transpose(0, 2, 1)
    flat_tab = table.reshape(BD * N, TD)

    outs = []
    for h in (0, 1):
        a_sl = slice(2 * h, 2 * h + 2)
        b_sl = slice(2 - 2 * h, 4 - 2 * h)
        idxT = _knn_call(knn_nT[a_sl], knn_n[b_sl], pcs[a_sl], pcsT[b_sl],
                         2 * h)
        idx = idxT.transpose(0, 2, 1)
        g = _sc_gather(flat_tab, idx.reshape(2 * N * K))
        grows = g.reshape(2, N * K, TD)
        outs.append(_mlp_calls(
            grows, p1s[a_sl], pcsT[a_sl], Wpos.T, bpos.reshape(1, C),
            gn0_gamma.reshape(1, C), gn0_beta.reshape(1, C), Wm.T,
            bm.reshape(1, C), gm_gamma.reshape(1, C), gm_beta.reshape(1, C)))

    o0 = outs[0].transpose(0, 2, 1)
    o1 = outs[1].transpose(0, 2, 1)
    return o0, o1
